# Initial kernel scaffold; baseline (speedup 1.0000x reference)
import jax
import jax.numpy as jnp
from jax.experimental import pallas as pl


def kernel(x, edge_index, W1, b1, W2, b2):
    raise NotImplementedError("write your pallas kernel here")


# speedup vs baseline: 15.2771x; 15.2771x over previous
import jax
import jax.numpy as jnp
from jax import lax
from jax.experimental import pallas as pl
from jax.experimental.pallas import tpu as pltpu
from jax.experimental.pallas import tpu_sc as plsc

_NC = 2
_NS = 16
_NW = _NC * _NS

_SB = 64
_NSUB = 8
_GB = _SB * _NSUB
_WB = 624

_PARAMS = pltpu.CompilerParams(use_tc_tiling_on_sc=False)


def _mesh():
    return plsc.VectorSubcoreMesh(core_axis_name="c", subcore_axis_name="s")


def _slabs(N):
    out = [(0, _WB)]
    tail0 = _NS * _WB
    if tail0 < N:
        out.append((tail0, N - tail0))
    return out


def _make_prop(N, E, H):
    assert E % _GB == 0
    ncht = E // _GB
    nch = (ncht + _NW - 1) // _NW
    assert _NS * _WB <= N <= _NS * _WB + _WB and (N - _NS * _WB) % 8 == 0

    def body(y_hbm, row_hbm, col_hbm, out_hbm, gidx, sidx, rows_v, acc_sh, sem):
        ci = lax.axis_index("c")
        si = lax.axis_index("s")
        wid = si * _NC + ci

        def zrow(i, _):
            for j in range(H // 16):
                rows_v[i, pl.ds(j * 16, 16)] = jnp.zeros((16,), jnp.float32)
            return _
        lax.fori_loop(0, _GB, zrow, None)
        for base, n in _slabs(N):
            if base > 0:
                @pl.when(si == 0)
                def _():
                    pltpu.sync_copy(rows_v.at[pl.ds(0, n)],
                                    acc_sh.at[pl.ds(base, n)])
            else:
                done = 0
                while done < _WB:
                    m = min(_GB, _WB - done)
                    off = pl.multiple_of(si * _WB + done, 8)
                    pltpu.sync_copy(rows_v.at[pl.ds(0, m)],
                                    acc_sh.at[pl.ds(off, m)])
                    done += m
        plsc.subcore_barrier()

        def chunk(t, _):
            c = t * _NW + wid

            @pl.when(c < ncht)
            def _():
                eoff = pl.multiple_of(c * _GB, 8)
                hs = [pltpu.async_copy(
                          row_hbm.at[pl.ds(eoff + j * _SB, _SB)], gidx[j], sem)
                      for j in range(_NSUB)]
                hs += [pltpu.async_copy(
                           col_hbm.at[pl.ds(eoff + j * _SB, _SB)], sidx[j], sem)
                       for j in range(_NSUB)]
                for h in hs:
                    h.wait()
                gs = [pltpu.async_copy(y_hbm.at[gidx[j]],
                                       rows_v.at[pl.ds(j * _SB, _SB)], sem)
                      for j in range(_NSUB)]
                for g in gs:
                    g.wait()
                for j in range(_NSUB):
                    pltpu.sync_copy(rows_v.at[pl.ds(j * _SB, _SB)],
                                    acc_sh.at[sidx[j]], add=True)
            return _
        lax.fori_loop(0, nch, chunk, None)
        plsc.subcore_barrier()

        for base, n in _slabs(N):
            if base > 0:
                @pl.when(si == 0)
                def _():
                    pltpu.sync_copy(acc_sh.at[pl.ds(base, n)],
                                    rows_v.at[pl.ds(0, n)])
                    pltpu.sync_copy(rows_v.at[pl.ds(0, n)],
                                    out_hbm.at[pl.ds(ci * N + base, n)])
            else:
                done = 0
                while done < _WB:
                    m = min(_GB, _WB - done)
                    off = pl.multiple_of(si * _WB + done, 8)
                    pltpu.sync_copy(acc_sh.at[pl.ds(off, m)],
                                    rows_v.at[pl.ds(0, m)])
                    pltpu.sync_copy(rows_v.at[pl.ds(0, m)],
                                    out_hbm.at[pl.ds(ci * N + off, m)])
                    done += m

    return pl.kernel(
        body,
        out_type=jax.ShapeDtypeStruct((_NC * N, H), jnp.float32),
        mesh=_mesh(),
        compiler_params=_PARAMS,
        scratch_types=[
            [pltpu.VMEM((_SB,), jnp.int32) for _ in range(_NSUB)],
            [pltpu.VMEM((_SB,), jnp.int32) for _ in range(_NSUB)],
            pltpu.VMEM((_GB, H), jnp.float32),
            pltpu.VMEM_SHARED((N, H), jnp.float32),
            pltpu.SemaphoreType.DMA,
        ],
    )


def _make_deg(N, E):
    H = 16
    assert E % _GB == 0
    ncht = E // _GB
    nch = (ncht + _NW - 1) // _NW

    def body(row_hbm, out_hbm, sidx, buf_v, acc_sh, sem):
        ci = lax.axis_index("c")
        si = lax.axis_index("s")
        wid = si * _NC + ci

        def zrow(i, _):
            buf_v[i, pl.ds(0, 16)] = jnp.zeros((16,), jnp.float32)
            return _
        lax.fori_loop(0, _WB, zrow, None)
        for base, n in _slabs(N):
            if base > 0:
                @pl.when(si == 0)
                def _():
                    pltpu.sync_copy(buf_v.at[pl.ds(0, n)],
                                    acc_sh.at[pl.ds(base, n)])
            else:
                off = pl.multiple_of(si * _WB, 8)
                pltpu.sync_copy(buf_v.at[pl.ds(0, _WB)],
                                acc_sh.at[pl.ds(off, _WB)])

        def onerow(i, _):
            buf_v[i, pl.ds(0, 16)] = jnp.ones((16,), jnp.float32)
            return _
        lax.fori_loop(0, _SB, onerow, None)
        plsc.subcore_barrier()

        def chunk(t, _):
            c = t * _NW + wid

            @pl.when(c < ncht)
            def _():
                eoff = pl.multiple_of(c * _GB, 8)
                for j in range(_NSUB):
                    pltpu.sync_copy(row_hbm.at[pl.ds(eoff + j * _SB, _SB)],
                                    sidx[j])
                for j in range(_NSUB):
                    pltpu.sync_copy(buf_v.at[pl.ds(0, _SB)],
                                    acc_sh.at[sidx[j]], add=True)
            return _
        lax.fori_loop(0, nch, chunk, None)
        plsc.subcore_barrier()

        for base, n in _slabs(N):
            if base > 0:
                @pl.when(si == 0)
                def _():
                    pltpu.sync_copy(acc_sh.at[pl.ds(base, n)],
                                    buf_v.at[pl.ds(0, n)])
                    pltpu.sync_copy(buf_v.at[pl.ds(0, n)],
                                    out_hbm.at[pl.ds(ci * N + base, n)])
            else:
                off = pl.multiple_of(si * _WB, 8)
                pltpu.sync_copy(acc_sh.at[pl.ds(off, _WB)],
                                buf_v.at[pl.ds(0, _WB)])
                pltpu.sync_copy(buf_v.at[pl.ds(0, _WB)],
                                out_hbm.at[pl.ds(ci * N + off, _WB)])

    return pl.kernel(
        body,
        out_type=jax.ShapeDtypeStruct((_NC * N, H), jnp.float32),
        mesh=_mesh(),
        compiler_params=_PARAMS,
        scratch_types=[
            [pltpu.VMEM((_SB,), jnp.int32) for _ in range(_NSUB)],
            pltpu.VMEM((_WB, H), jnp.float32),
            pltpu.VMEM_SHARED((N, H), jnp.float32),
            pltpu.SemaphoreType.DMA,
        ],
    )


def kernel(x, edge_index, W1, b1, W2, b2):
    N, D = x.shape
    E = edge_index.shape[1]
    K1, _, H = W1.shape
    row, col = edge_index[0], edge_index[1]

    deg_call = _make_deg(N, E)
    part = deg_call(row)
    deg = part[:N, 0] + part[N:, 0]
    dis = jnp.where(deg > 0, lax.rsqrt(deg), 0.0)[:, None]

    prop_call = _make_prop(N, E, H)

    def lhat(b):
        p = prop_call(dis * b, row, col)
        return -dis * (p[:N] + p[N:])

    z = jnp.einsum("nd,kdh->knh", x, W1)
    bk1 = z[K1 - 1]
    bk2 = jnp.zeros_like(bk1)
    for k in range(K1 - 2, 0, -1):
        bk = z[k] + 2.0 * lhat(bk1) - bk2
        bk1, bk2 = bk, bk1
    h = z[0] + lhat(bk1) - bk2 + b1

    h = jax.nn.relu(h)
    out = h @ W2[0] + b2
    return jax.nn.softmax(out, axis=1)


# speedup vs baseline: 19.1435x; 1.2531x over previous
import jax
import jax.numpy as jnp
from jax import lax
from jax.experimental import pallas as pl
from jax.experimental.pallas import tpu as pltpu
from jax.experimental.pallas import tpu_sc as plsc

_NC = 2
_NS = 16
_NW = _NC * _NS

_SB = 64
_NSUB = 8
_GB = _SB * _NSUB
_WB = 624

_PARAMS = pltpu.CompilerParams(use_tc_tiling_on_sc=False)


def _mesh():
    return plsc.VectorSubcoreMesh(core_axis_name="c", subcore_axis_name="s")


def _slabs(N):
    out = [(0, _WB)]
    tail0 = _NS * _WB
    if tail0 < N:
        out.append((tail0, N - tail0))
    return out


def _make_prop(N, E, H):
    assert E % _GB == 0
    ncht = E // _GB
    nch = (ncht + _NW - 1) // _NW
    assert _NS * _WB <= N <= _NS * _WB + _WB and (N - _NS * _WB) % 8 == 0

    nbuf = 2
    nu = (nch + nbuf - 1) // nbuf

    def body(y_hbm, row_hbm, col_hbm, out_hbm, gidx, sidx, rows_v, acc_sh,
             sem, sems):
        ci = lax.axis_index("c")
        si = lax.axis_index("s")
        wid = si * _NC + ci

        def zrow(i, _):
            for j in range(H // 16):
                rows_v[0][i, pl.ds(j * 16, 16)] = jnp.zeros((16,), jnp.float32)
            return _
        lax.fori_loop(0, _GB, zrow, None)
        for base, n in _slabs(N):
            if base > 0:
                @pl.when(si == 0)
                def _():
                    pltpu.sync_copy(rows_v[0].at[pl.ds(0, n)],
                                    acc_sh.at[pl.ds(base, n)])
            else:
                done = 0
                while done < _WB:
                    m = min(_GB, _WB - done)
                    off = pl.multiple_of(si * _WB + done, 8)
                    pltpu.sync_copy(rows_v[0].at[pl.ds(0, m)],
                                    acc_sh.at[pl.ds(off, m)])
                    done += m
        plsc.subcore_barrier()

        def drain(b, pred):
            @pl.when(pred)
            def _():
                for j in range(_NSUB):
                    pltpu.make_async_copy(
                        rows_v[b].at[pl.ds(j * _SB, _SB)],
                        acc_sh.at[sidx[b][j]], sems[b]).wait()

        def step(u, _):
            for b in range(nbuf):
                c = (u * nbuf + b) * _NW + wid
                cprev = c - nbuf * _NW
                drain(b, (u > 0) & (cprev < ncht))

                @pl.when(c < ncht)
                def _():
                    eoff = pl.multiple_of(c * _GB, 8)
                    hs = [pltpu.async_copy(
                              row_hbm.at[pl.ds(eoff + j * _SB, _SB)],
                              gidx[b][j], sem)
                          for j in range(_NSUB)]
                    hs += [pltpu.async_copy(
                               col_hbm.at[pl.ds(eoff + j * _SB, _SB)],
                               sidx[b][j], sem)
                           for j in range(_NSUB)]
                    for h in hs:
                        h.wait()
                    gs = [pltpu.async_copy(y_hbm.at[gidx[b][j]],
                                           rows_v[b].at[pl.ds(j * _SB, _SB)],
                                           sem)
                          for j in range(_NSUB)]
                    for g in gs:
                        g.wait()
                    for j in range(_NSUB):
                        pltpu.async_copy(rows_v[b].at[pl.ds(j * _SB, _SB)],
                                         acc_sh.at[sidx[b][j]], sems[b],
                                         add=True)
            return _
        lax.fori_loop(0, nu, step, None)
        for b in range(nbuf):
            clast = ((nu - 1) * nbuf + b) * _NW + wid
            drain(b, clast < ncht)
        plsc.subcore_barrier()

        for base, n in _slabs(N):
            if base > 0:
                @pl.when(si == 0)
                def _():
                    pltpu.sync_copy(acc_sh.at[pl.ds(base, n)],
                                    rows_v[0].at[pl.ds(0, n)])
                    pltpu.sync_copy(rows_v[0].at[pl.ds(0, n)],
                                    out_hbm.at[pl.ds(ci * N + base, n)])
            else:
                done = 0
                while done < _WB:
                    m = min(_GB, _WB - done)
                    off = pl.multiple_of(si * _WB + done, 8)
                    pltpu.sync_copy(acc_sh.at[pl.ds(off, m)],
                                    rows_v[0].at[pl.ds(0, m)])
                    pltpu.sync_copy(rows_v[0].at[pl.ds(0, m)],
                                    out_hbm.at[pl.ds(ci * N + off, m)])
                    done += m

    return pl.kernel(
        body,
        out_type=jax.ShapeDtypeStruct((_NC * N, H), jnp.float32),
        mesh=_mesh(),
        compiler_params=_PARAMS,
        scratch_types=[
            [[pltpu.VMEM((_SB,), jnp.int32) for _ in range(_NSUB)]
             for _ in range(nbuf)],
            [[pltpu.VMEM((_SB,), jnp.int32) for _ in range(_NSUB)]
             for _ in range(nbuf)],
            [pltpu.VMEM((_GB, H), jnp.float32) for _ in range(nbuf)],
            pltpu.VMEM_SHARED((N, H), jnp.float32),
            pltpu.SemaphoreType.DMA,
            [pltpu.SemaphoreType.DMA for _ in range(nbuf)],
        ],
    )


def _make_deg(N, E):
    H = 16
    assert E % _GB == 0
    ncht = E // _GB
    nch = (ncht + _NW - 1) // _NW

    def body(row_hbm, out_hbm, sidx, buf_v, acc_sh, sem):
        ci = lax.axis_index("c")
        si = lax.axis_index("s")
        wid = si * _NC + ci

        def zrow(i, _):
            buf_v[i, pl.ds(0, 16)] = jnp.zeros((16,), jnp.float32)
            return _
        lax.fori_loop(0, _WB, zrow, None)
        for base, n in _slabs(N):
            if base > 0:
                @pl.when(si == 0)
                def _():
                    pltpu.sync_copy(buf_v.at[pl.ds(0, n)],
                                    acc_sh.at[pl.ds(base, n)])
            else:
                off = pl.multiple_of(si * _WB, 8)
                pltpu.sync_copy(buf_v.at[pl.ds(0, _WB)],
                                acc_sh.at[pl.ds(off, _WB)])

        def onerow(i, _):
            buf_v[i, pl.ds(0, 16)] = jnp.ones((16,), jnp.float32)
            return _
        lax.fori_loop(0, _SB, onerow, None)
        plsc.subcore_barrier()

        def chunk(t, _):
            c = t * _NW + wid

            @pl.when(c < ncht)
            def _():
                eoff = pl.multiple_of(c * _GB, 8)
                for j in range(_NSUB):
                    pltpu.sync_copy(row_hbm.at[pl.ds(eoff + j * _SB, _SB)],
                                    sidx[j])
                for j in range(_NSUB):
                    pltpu.sync_copy(buf_v.at[pl.ds(0, _SB)],
                                    acc_sh.at[sidx[j]], add=True)
            return _
        lax.fori_loop(0, nch, chunk, None)
        plsc.subcore_barrier()

        for base, n in _slabs(N):
            if base > 0:
                @pl.when(si == 0)
                def _():
                    pltpu.sync_copy(acc_sh.at[pl.ds(base, n)],
                                    buf_v.at[pl.ds(0, n)])
                    pltpu.sync_copy(buf_v.at[pl.ds(0, n)],
                                    out_hbm.at[pl.ds(ci * N + base, n)])
            else:
                off = pl.multiple_of(si * _WB, 8)
                pltpu.sync_copy(acc_sh.at[pl.ds(off, _WB)],
                                buf_v.at[pl.ds(0, _WB)])
                pltpu.sync_copy(buf_v.at[pl.ds(0, _WB)],
                                out_hbm.at[pl.ds(ci * N + off, _WB)])

    return pl.kernel(
        body,
        out_type=jax.ShapeDtypeStruct((_NC * N, H), jnp.float32),
        mesh=_mesh(),
        compiler_params=_PARAMS,
        scratch_types=[
            [pltpu.VMEM((_SB,), jnp.int32) for _ in range(_NSUB)],
            pltpu.VMEM((_WB, H), jnp.float32),
            pltpu.VMEM_SHARED((N, H), jnp.float32),
            pltpu.SemaphoreType.DMA,
        ],
    )


def kernel(x, edge_index, W1, b1, W2, b2):
    N, D = x.shape
    E = edge_index.shape[1]
    K1, _, H = W1.shape
    row, col = edge_index[0], edge_index[1]

    deg_call = _make_deg(N, E)
    part = deg_call(row)
    deg = part[:N, 0] + part[N:, 0]
    dis = jnp.where(deg > 0, lax.rsqrt(deg), 0.0)[:, None]

    prop_call = _make_prop(N, E, H)

    def lhat(b):
        p = prop_call(dis * b, row, col)
        return -dis * (p[:N] + p[N:])

    z = jnp.einsum("nd,kdh->knh", x, W1)
    bk1 = z[K1 - 1]
    bk2 = jnp.zeros_like(bk1)
    for k in range(K1 - 2, 0, -1):
        bk = z[k] + 2.0 * lhat(bk1) - bk2
        bk1, bk2 = bk, bk1
    h = z[0] + lhat(bk1) - bk2 + b1

    h = jax.nn.relu(h)
    out = h @ W2[0] + b2
    return jax.nn.softmax(out, axis=1)
